# baseline (speedup 1.0000x reference)
import functools

import jax
import jax.numpy as jnp
from jax import lax
from jax.experimental import pallas as pl
from jax.experimental.pallas import tpu as pltpu
from jax.experimental.pallas import tpu_sc as plsc

SEG_LEN = 50
NUM_SAMPLES = 20
B = 128
N = 8192
VS = N - SEG_LEN

_NC = 2
_NS = 16
_NW = _NC * _NS
_SEGS = NUM_SAMPLES * B
_SPW = _SEGS // _NW
_TROWS = B * N * 2 // 128
_SOFF = 128

_SUB = 64
_LANE = 128


_BB = 8


def _tc_body(xp_ref, g_ref, w1_ref, b1_ref, w2_ref, b2_ref, w3_ref, b3_ref,
             starts_ref, h1_ref):
    rows = []
    for r in range(_BB):
        xp = xp_ref[r]

        j4 = lax.broadcasted_iota(jnp.int32, (2, N + 4), 1)
        xz = jnp.where((j4 >= 2) & (j4 < N + 2), xp, 0.0)
        x10 = jnp.concatenate([xz[:, k:k + N] for k in range(5)], axis=0)
        h1 = jnp.dot(w1_ref[...], x10, preferred_element_type=jnp.float32)
        h1 = jnp.maximum(h1 + b1_ref[...], 0.0)

        h1_ref[:, 0:1] = jnp.zeros((64, 1), jnp.float32)
        h1_ref[:, 1:N + 1] = h1
        h1_ref[:, N + 1:N + 2] = jnp.zeros((64, 1), jnp.float32)
        acc = jnp.dot(w2_ref[0], h1_ref[:, 0:N],
                      preferred_element_type=jnp.float32)
        acc += jnp.dot(w2_ref[1], h1_ref[:, 1:N + 1],
                       preferred_element_type=jnp.float32)
        acc += jnp.dot(w2_ref[2], h1_ref[:, 2:N + 2],
                       preferred_element_type=jnp.float32)
        h2 = jnp.maximum(acc + b2_ref[...], 0.0)

        imp = jax.nn.sigmoid(jnp.sum(h2 * w3_ref[...], axis=0, keepdims=True)
                             + b3_ref[...])

        fwd = xp[:, 3:N + 3] - xp[:, 2:N + 2]
        bwd = xp[:, 2:N + 2] - xp[:, 1:N + 1]
        cross = fwd[0:1] * bwd[1:2] - fwd[1:2] * bwd[0:1]
        rows.append(imp + 0.3 * jnp.abs(cross))

    comb = jnp.concatenate(rows, axis=0)
    lin = lax.broadcasted_iota(jnp.int32, (_BB, N), 1)
    l128 = lax.broadcasted_iota(jnp.int32, (_BB, _LANE), 1)
    sv = jnp.zeros((_BB, _LANE), jnp.int32)
    for i in range(NUM_SAMPLES):
        score = comb + g_ref[i]
        m = jnp.max(score, axis=1, keepdims=True)
        start = jnp.min(jnp.where(score == m, lin, jnp.int32(2 ** 30)),
                        axis=1, keepdims=True)
        sv = sv + jnp.where(l128 == i, start, 0)
        seg_mask = (lin >= start) & (lin < start + SEG_LEN)
        comb = jnp.where(seg_mask, comb * 0.1, comb)
    starts_ref[...] = sv


def _tc_starts(xp, gp, w1r, b1c, w2r, b2c, w3c, b3c):
    grid = (B // _BB,)
    return pl.pallas_call(
        _tc_body,
        grid=grid,
        in_specs=[
            pl.BlockSpec((_BB, 2, N + 4), lambda b: (b, 0, 0)),
            pl.BlockSpec((NUM_SAMPLES, _BB, N), lambda b: (0, b, 0)),
            pl.BlockSpec((64, 10), lambda b: (0, 0)),
            pl.BlockSpec((64, 1), lambda b: (0, 0)),
            pl.BlockSpec((3, 32, 64), lambda b: (0, 0, 0)),
            pl.BlockSpec((32, 1), lambda b: (0, 0)),
            pl.BlockSpec((32, 1), lambda b: (0, 0)),
            pl.BlockSpec((1, 1), lambda b: (0, 0)),
        ],
        out_specs=pl.BlockSpec((_BB, _LANE), lambda b: (b, 0)),
        out_shape=jax.ShapeDtypeStruct((B, _LANE), jnp.int32),
        scratch_shapes=[pltpu.VMEM((64, N + 2), jnp.float32)],
    )(xp, gp, w1r, b1c, w2r, b2c, w3c, b3c)


def _sc_gather_body(pts_hbm, idx_hbm, ext_hbm, out_hbm,
                    idx_v, ext_v, stage, rows_v, sem):
    wid = lax.axis_index("s") * _NC + lax.axis_index("c")
    base = wid * _SPW
    pltpu.sync_copy(idx_hbm.at[wid], idx_v)
    pltpu.sync_copy(ext_hbm.at[pl.ds(base, _SPW)], ext_v)
    cp0 = pltpu.async_copy(pts_hbm.at[idx_v.at[0]], stage.at[pl.ds(0, 128)],
                           sem)
    cp1 = pltpu.async_copy(pts_hbm.at[idx_v.at[1]],
                           stage.at[pl.ds(_SOFF, 128)], sem)
    cp0.wait()
    cp1.wait()

    def seg_body(s, _):
        for c in range(8):
            lin = ext_v[s, pl.ds(c * 16, 16)]
            er = lax.shift_right_logical(lin, 7)
            ec = lax.bitwise_and(lin, 127)
            rows_v[s, pl.ds(c * 16, 16)] = plsc.load_gather(stage, [er, ec])
        return 0

    lax.fori_loop(0, _SPW, seg_body, 0)
    pltpu.sync_copy(rows_v, out_hbm.at[pl.ds(base, _SPW)])


@functools.cache
def _sc_gather_kernel():
    return pl.kernel(
        _sc_gather_body,
        mesh=plsc.VectorSubcoreMesh(core_axis_name="c", subcore_axis_name="s"),
        out_type=jax.ShapeDtypeStruct((_SEGS, 128), jnp.float32),
        scratch_types=[
            pltpu.VMEM((2, 128), jnp.int32),
            pltpu.VMEM((_SPW, 128), jnp.int32),
            pltpu.VMEM((2 * _SOFF, 128), jnp.float32),
            pltpu.VMEM((_SPW, 128), jnp.float32),
            pltpu.SemaphoreType.DMA,
        ],
        compiler_params=pltpu.CompilerParams(needs_layout_passes=False),
    )


def kernel(points, w1, b1, w2, b2, w3, b3):
    skey = jax.random.key(42)
    keys = jax.vmap(lambda i: jax.random.fold_in(skey, i))(
        jnp.arange(NUM_SAMPLES))
    g = jax.vmap(lambda k: jax.random.gumbel(k, (B, VS), jnp.float32))(keys)
    gp = jnp.pad(g, ((0, 0), (0, 0), (0, SEG_LEN)), constant_values=-1e30)

    xt = jnp.transpose(points, (0, 2, 1))
    xp = jnp.concatenate([xt[:, :, -2:], xt, xt[:, :, :2]], axis=2)

    w1r = jnp.transpose(w1, (0, 2, 1)).reshape(64, 10)
    w2r = jnp.transpose(w2, (2, 0, 1))
    b1c = b1.reshape(64, 1)
    b2c = b2.reshape(32, 1)
    w3c = w3.reshape(32, 1)
    b3c = b3.reshape(1, 1)

    starts2 = _tc_starts(xp, gp, w1r, b1c, w2r, b2c, w3c, b3c)
    starts = starts2[:, :NUM_SAMPLES].T

    bcol = jnp.arange(B, dtype=jnp.int32) * 128
    g0 = (bcol[None, :] + (starts // 64)).reshape(_SEGS)
    g1 = jnp.minimum(g0 + 1, _TROWS - 1)
    idxw = jnp.stack([g0.reshape(_NW, _SPW), g1.reshape(_NW, _SPW)],
                     axis=1)
    idxh = jnp.pad(idxw, ((0, 0), (0, 0), (0, 128 - _SPW)), mode="edge")

    off = (2 * (starts % 64)).reshape(_SEGS)
    s_local = jnp.arange(_SEGS, dtype=jnp.int32) % _SPW
    j = jnp.arange(128, dtype=jnp.int32)
    oj = off[:, None] + j[None, :]
    ext = 128 * s_local[:, None] + oj + jnp.where(oj >= 128, 127 * 128, 0)

    pts128 = points.reshape(_TROWS, 128)
    out = _sc_gather_kernel()(pts128, idxh, ext)
    return out[:, :100].reshape(NUM_SAMPLES, B, SEG_LEN, 2)


# speedup vs baseline: 1.6685x; 1.6685x over previous
import functools

import jax
import jax.numpy as jnp
from jax import lax
from jax.experimental import pallas as pl
from jax.experimental.pallas import tpu as pltpu
from jax.experimental.pallas import tpu_sc as plsc

SEG_LEN = 50
NUM_SAMPLES = 20
B = 128
N = 8192
VS = N - SEG_LEN

_NC = 2
_NS = 16
_NW = _NC * _NS
_SEGS = NUM_SAMPLES * B
_SPW = _SEGS // _NW
_TROWS = B * N * 2 // 128
_SOFF = 128

_SUB = 64
_LANE = 128


_BB = 8


def _tc_body(xp_ref, g_ref, w1_ref, b1_ref, w2_ref, b2_ref, w3_ref, b3_ref,
             starts_ref, tbl_ref, h1_ref):
    rows = []
    for r in range(_BB):
        xp = xp_ref[r]

        tbl_ref[pl.ds(r * 128, 128)] = jnp.reshape(xp[:, 2:N + 2], (128, 128))

        j4 = lax.broadcasted_iota(jnp.int32, (2, N + 4), 1)
        xz = jnp.where((j4 >= 2) & (j4 < N + 2), xp, 0.0)
        x10 = jnp.concatenate([xz[:, k:k + N] for k in range(5)], axis=0)
        h1 = jnp.dot(w1_ref[...], x10, preferred_element_type=jnp.float32)
        h1 = jnp.maximum(h1 + b1_ref[...], 0.0)

        h1_ref[:, 0:1] = jnp.zeros((64, 1), jnp.float32)
        h1_ref[:, 1:N + 1] = h1
        h1_ref[:, N + 1:N + 2] = jnp.zeros((64, 1), jnp.float32)
        acc = jnp.dot(w2_ref[0], h1_ref[:, 0:N],
                      preferred_element_type=jnp.float32)
        acc += jnp.dot(w2_ref[1], h1_ref[:, 1:N + 1],
                       preferred_element_type=jnp.float32)
        acc += jnp.dot(w2_ref[2], h1_ref[:, 2:N + 2],
                       preferred_element_type=jnp.float32)
        h2 = jnp.maximum(acc + b2_ref[...], 0.0)

        imp = jax.nn.sigmoid(jnp.sum(h2 * w3_ref[...], axis=0, keepdims=True)
                             + b3_ref[...])

        fwd = xp[:, 3:N + 3] - xp[:, 2:N + 2]
        bwd = xp[:, 2:N + 2] - xp[:, 1:N + 1]
        cross = fwd[0:1] * bwd[1:2] - fwd[1:2] * bwd[0:1]
        rows.append(imp + 0.3 * jnp.abs(cross))

    comb = jnp.concatenate(rows, axis=0)
    lin = lax.broadcasted_iota(jnp.int32, (_BB, N), 1)
    l128 = lax.broadcasted_iota(jnp.int32, (_BB, _LANE), 1)
    sv = jnp.zeros((_BB, _LANE), jnp.int32)
    for i in range(NUM_SAMPLES):
        score = comb + g_ref[i]
        m = jnp.max(score, axis=1, keepdims=True)
        start = jnp.min(jnp.where(score == m, lin, jnp.int32(2 ** 30)),
                        axis=1, keepdims=True)
        sv = sv + jnp.where(l128 == i, start, 0)
        seg_mask = (lin >= start) & (lin < start + SEG_LEN)
        comb = jnp.where(seg_mask, comb * 0.1, comb)
    starts_ref[...] = sv


def _tc_starts(xp, gp, w1r, b1c, w2r, b2c, w3c, b3c):
    grid = (B // _BB,)
    return pl.pallas_call(
        _tc_body,
        grid=grid,
        in_specs=[
            pl.BlockSpec((_BB, 2, N + 4), lambda b: (b, 0, 0)),
            pl.BlockSpec((NUM_SAMPLES, _BB, N), lambda b: (0, b, 0)),
            pl.BlockSpec((64, 10), lambda b: (0, 0)),
            pl.BlockSpec((64, 1), lambda b: (0, 0)),
            pl.BlockSpec((3, 32, 64), lambda b: (0, 0, 0)),
            pl.BlockSpec((32, 1), lambda b: (0, 0)),
            pl.BlockSpec((32, 1), lambda b: (0, 0)),
            pl.BlockSpec((1, 1), lambda b: (0, 0)),
        ],
        out_specs=[
            pl.BlockSpec((_BB, _LANE), lambda b: (b, 0)),
            pl.BlockSpec((_BB * 128, 128), lambda b: (b, 0)),
        ],
        out_shape=[
            jax.ShapeDtypeStruct((B, _LANE), jnp.int32),
            jax.ShapeDtypeStruct((_TROWS, 128), jnp.float32),
        ],
        scratch_shapes=[pltpu.VMEM((64, N + 2), jnp.float32)],
    )(xp, gp, w1r, b1c, w2r, b2c, w3c, b3c)


def _sc_gather_body(pts_hbm, idx_hbm, ext_hbm, out_hbm,
                    idx_v, ext_v, stage, rows_v, sem):
    wid = lax.axis_index("s") * _NC + lax.axis_index("c")
    base = wid * _SPW
    pltpu.sync_copy(idx_hbm.at[wid], idx_v)
    pltpu.sync_copy(ext_hbm.at[pl.ds(base, _SPW)], ext_v)
    cps = [
        pltpu.async_copy(pts_hbm.at[idx_v.at[bank]],
                         stage.at[pl.ds(bank * _SOFF, 128)], sem)
        for bank in range(4)
    ]
    for cp in cps:
        cp.wait()

    def seg_body(s, _):
        for c in range(8):
            lin = ext_v[s, pl.ds(c * 16, 16)]
            er = lax.shift_right_logical(lin, 7)
            ec = lax.bitwise_and(lin, 127)
            rows_v[s, pl.ds(c * 16, 16)] = plsc.load_gather(stage, [er, ec])
        return 0

    lax.fori_loop(0, _SPW, seg_body, 0)
    pltpu.sync_copy(rows_v, out_hbm.at[pl.ds(base, _SPW)])


@functools.cache
def _sc_gather_kernel():
    return pl.kernel(
        _sc_gather_body,
        mesh=plsc.VectorSubcoreMesh(core_axis_name="c", subcore_axis_name="s"),
        out_type=jax.ShapeDtypeStruct((_SEGS, 128), jnp.float32),
        scratch_types=[
            pltpu.VMEM((4, 128), jnp.int32),
            pltpu.VMEM((_SPW, 128), jnp.int32),
            pltpu.VMEM((4 * _SOFF, 128), jnp.float32),
            pltpu.VMEM((_SPW, 128), jnp.float32),
            pltpu.SemaphoreType.DMA,
        ],
        compiler_params=pltpu.CompilerParams(needs_layout_passes=False),
    )


def kernel(points, w1, b1, w2, b2, w3, b3):
    skey = jax.random.key(42)
    keys = jax.vmap(lambda i: jax.random.fold_in(skey, i))(
        jnp.arange(NUM_SAMPLES))
    g = jax.vmap(lambda k: jax.random.gumbel(k, (B, VS), jnp.float32))(keys)
    gp = jnp.pad(g, ((0, 0), (0, 0), (0, SEG_LEN)), constant_values=-1e30)

    xt = jnp.transpose(points, (0, 2, 1))
    xp = jnp.concatenate([xt[:, :, -2:], xt, xt[:, :, :2]], axis=2)

    w1r = jnp.transpose(w1, (0, 2, 1)).reshape(64, 10)
    w2r = jnp.transpose(w2, (2, 0, 1))
    b1c = b1.reshape(64, 1)
    b2c = b2.reshape(32, 1)
    w3c = w3.reshape(32, 1)
    b3c = b3.reshape(1, 1)

    starts2, table = _tc_starts(xp, gp, w1r, b1c, w2r, b2c, w3c, b3c)
    starts = starts2[:, :NUM_SAMPLES].T

    sflat = starts.reshape(_SEGS)
    bno = jnp.tile(jnp.arange(B, dtype=jnp.int32), (NUM_SAMPLES,))
    blk = sflat // 128
    o = sflat % 128
    need2 = (o >= 128 - SEG_LEN + 1).astype(jnp.int32)
    xr0 = bno * 128 + blk
    banks = jnp.stack([xr0, xr0 + need2, xr0 + 64, xr0 + 64 + need2],
                      axis=0)
    idxw = banks.reshape(4, _NW, _SPW).transpose(1, 0, 2)
    idxh = jnp.pad(idxw, ((0, 0), (0, 0), (0, 128 - _SPW)), mode="edge")

    s_local = jnp.arange(_SEGS, dtype=jnp.int32) % _SPW
    j = jnp.minimum(jnp.arange(128, dtype=jnp.int32), 99)
    p = j // 2
    c = j & 1
    pos = o[:, None] + p[None, :]
    bank = 2 * c[None, :] + (pos >= 128).astype(jnp.int32)
    ext = 128 * (128 * bank + s_local[:, None]) + (pos & 127)

    out = _sc_gather_kernel()(table, idxh, ext)
    return out[:, :100].reshape(NUM_SAMPLES, B, SEG_LEN, 2)


# speedup vs baseline: 2.1142x; 1.2671x over previous
import functools

import jax
import jax.numpy as jnp
from jax import lax
from jax.experimental import pallas as pl
from jax.experimental.pallas import tpu as pltpu
from jax.experimental.pallas import tpu_sc as plsc

SEG_LEN = 50
NUM_SAMPLES = 20
B = 128
N = 8192
VS = N - SEG_LEN

_NC = 2
_NS = 16
_NW = _NC * _NS
_SEGS = NUM_SAMPLES * B
_SPW = _SEGS // _NW
_TROWS = B * N * 2 // 128
_SOFF = 128

_SUB = 64
_LANE = 128


_BB = 8


def _tc_body(xp_ref, g_ref, w1_ref, b1_ref, w2_ref, b2_ref, w3_ref, b3_ref,
             starts_ref, tbl_ref):
    rows = []
    for r in range(_BB):
        xp = xp_ref[r]

        tbl_ref[pl.ds(r * 128, 128)] = jnp.reshape(xp[:, 2:N + 2], (128, 128))

        j4 = lax.broadcasted_iota(jnp.int32, (2, N + 4), 1)
        xz = jnp.where((j4 >= 2) & (j4 < N + 2), xp, 0.0)
        x10 = jnp.concatenate([xz[:, k:k + N] for k in range(5)], axis=0)
        h1 = jnp.dot(w1_ref[...], x10, preferred_element_type=jnp.float32)
        h1 = jnp.maximum(h1 + b1_ref[...], 0.0)

        p0 = jnp.dot(w2_ref[0], h1, preferred_element_type=jnp.float32)
        p1 = jnp.dot(w2_ref[1], h1, preferred_element_type=jnp.float32)
        p2 = jnp.dot(w2_ref[2], h1, preferred_element_type=jnp.float32)
        ln = lax.broadcasted_iota(jnp.int32, (32, N), 1)
        acc = (p1
               + jnp.where(ln == 0, 0.0, pltpu.roll(p0, 1, 1))
               + jnp.where(ln == N - 1, 0.0, pltpu.roll(p2, N - 1, 1)))
        h2 = jnp.maximum(acc + b2_ref[...], 0.0)

        imp = jax.nn.sigmoid(jnp.dot(w3_ref[...], h2,
                                     preferred_element_type=jnp.float32)
                             + b3_ref[...])

        fwd = xp[:, 3:N + 3] - xp[:, 2:N + 2]
        bwd = xp[:, 2:N + 2] - xp[:, 1:N + 1]
        cross = fwd[0:1] * bwd[1:2] - fwd[1:2] * bwd[0:1]
        rows.append(imp + 0.3 * jnp.abs(cross))

    comb = jnp.concatenate(rows, axis=0)
    lin = lax.broadcasted_iota(jnp.int32, (_BB, N), 1)
    linv = lax.broadcasted_iota(jnp.int32, (_BB, VS), 1)
    l128 = lax.broadcasted_iota(jnp.int32, (_BB, _LANE), 1)
    sv = jnp.zeros((_BB, _LANE), jnp.int32)
    for i in range(NUM_SAMPLES):
        score = comb[:, :VS] + g_ref[i]
        m = jnp.max(score, axis=1, keepdims=True)
        start = jnp.min(jnp.where(score == m, linv, jnp.int32(2 ** 30)),
                        axis=1, keepdims=True)
        sv = sv + jnp.where(l128 == i, start, 0)
        seg_mask = (lin >= start) & (lin < start + SEG_LEN)
        comb = jnp.where(seg_mask, comb * 0.1, comb)
    starts_ref[...] = sv


def _tc_starts(xp, gp, w1r, b1c, w2r, b2c, w3c, b3c):
    grid = (B // _BB,)
    return pl.pallas_call(
        _tc_body,
        grid=grid,
        in_specs=[
            pl.BlockSpec((_BB, 2, N + 4), lambda b: (b, 0, 0)),
            pl.BlockSpec((NUM_SAMPLES, _BB, VS), lambda b: (0, b, 0)),
            pl.BlockSpec((64, 10), lambda b: (0, 0)),
            pl.BlockSpec((64, 1), lambda b: (0, 0)),
            pl.BlockSpec((3, 32, 64), lambda b: (0, 0, 0)),
            pl.BlockSpec((32, 1), lambda b: (0, 0)),
            pl.BlockSpec((1, 32), lambda b: (0, 0)),
            pl.BlockSpec((1, 1), lambda b: (0, 0)),
        ],
        out_specs=[
            pl.BlockSpec((_BB, _LANE), lambda b: (b, 0)),
            pl.BlockSpec((_BB * 128, 128), lambda b: (b, 0)),
        ],
        out_shape=[
            jax.ShapeDtypeStruct((B, _LANE), jnp.int32),
            jax.ShapeDtypeStruct((_TROWS, 128), jnp.float32),
        ],
    )(xp, gp, w1r, b1c, w2r, b2c, w3c, b3c)


def _sc_gather_body(pts_hbm, idx_hbm, ext_hbm, out_hbm,
                    idx_v, ext_v, stage, rows_v, sem):
    wid = lax.axis_index("s") * _NC + lax.axis_index("c")
    base = wid * _SPW
    pltpu.sync_copy(idx_hbm.at[wid], idx_v)
    pltpu.sync_copy(ext_hbm.at[pl.ds(base, _SPW)], ext_v)
    cps = [
        pltpu.async_copy(pts_hbm.at[idx_v.at[bank]],
                         stage.at[pl.ds(bank * _SOFF, 128)], sem)
        for bank in range(4)
    ]
    for cp in cps:
        cp.wait()

    def seg_body(s, _):
        for c in range(8):
            lin = ext_v[s, pl.ds(c * 16, 16)]
            er = lax.shift_right_logical(lin, 7)
            ec = lax.bitwise_and(lin, 127)
            rows_v[s, pl.ds(c * 16, 16)] = plsc.load_gather(stage, [er, ec])
        return 0

    lax.fori_loop(0, _SPW, seg_body, 0)
    pltpu.sync_copy(rows_v, out_hbm.at[pl.ds(base, _SPW)])


@functools.cache
def _sc_gather_kernel():
    return pl.kernel(
        _sc_gather_body,
        mesh=plsc.VectorSubcoreMesh(core_axis_name="c", subcore_axis_name="s"),
        out_type=jax.ShapeDtypeStruct((_SEGS, 128), jnp.float32),
        scratch_types=[
            pltpu.VMEM((4, 128), jnp.int32),
            pltpu.VMEM((_SPW, 128), jnp.int32),
            pltpu.VMEM((4 * _SOFF, 128), jnp.float32),
            pltpu.VMEM((_SPW, 128), jnp.float32),
            pltpu.SemaphoreType.DMA,
        ],
        compiler_params=pltpu.CompilerParams(needs_layout_passes=False),
    )


def kernel(points, w1, b1, w2, b2, w3, b3):
    skey = jax.random.key(42)
    keys = jax.vmap(lambda i: jax.random.fold_in(skey, i))(
        jnp.arange(NUM_SAMPLES))
    gp = jax.vmap(lambda k: jax.random.gumbel(k, (B, VS), jnp.float32))(keys)

    xt = jnp.transpose(points, (0, 2, 1))
    xp = jnp.concatenate([xt[:, :, -2:], xt, xt[:, :, :2]], axis=2)

    w1r = jnp.transpose(w1, (0, 2, 1)).reshape(64, 10)
    w2r = jnp.transpose(w2, (2, 0, 1))
    b1c = b1.reshape(64, 1)
    b2c = b2.reshape(32, 1)
    w3c = w3.reshape(1, 32)
    b3c = b3.reshape(1, 1)

    starts2, table = _tc_starts(xp, gp, w1r, b1c, w2r, b2c, w3c, b3c)
    starts = starts2[:, :NUM_SAMPLES].T

    sflat = starts.reshape(_SEGS)
    bno = jnp.tile(jnp.arange(B, dtype=jnp.int32), (NUM_SAMPLES,))
    blk = sflat // 128
    o = sflat % 128
    need2 = (o >= 128 - SEG_LEN + 1).astype(jnp.int32)
    xr0 = bno * 128 + blk
    banks = jnp.stack([xr0, xr0 + need2, xr0 + 64, xr0 + 64 + need2],
                      axis=0)
    idxw = banks.reshape(4, _NW, _SPW).transpose(1, 0, 2)
    idxh = jnp.pad(idxw, ((0, 0), (0, 0), (0, 128 - _SPW)), mode="edge")

    s_local = jnp.arange(_SEGS, dtype=jnp.int32) % _SPW
    j = jnp.minimum(jnp.arange(128, dtype=jnp.int32), 99)
    p = j // 2
    c = j & 1
    pos = o[:, None] + p[None, :]
    bank = 2 * c[None, :] + (pos >= 128).astype(jnp.int32)
    ext = 128 * (128 * bank + s_local[:, None]) + (pos & 127)

    out = _sc_gather_kernel()(table, idxh, ext)
    return out[:, :100].reshape(NUM_SAMPLES, B, SEG_LEN, 2)


# speedup vs baseline: 2.1494x; 1.0167x over previous
import functools

import jax
import jax.numpy as jnp
from jax import lax
from jax.experimental import pallas as pl
from jax.experimental.pallas import tpu as pltpu
from jax.experimental.pallas import tpu_sc as plsc

SEG_LEN = 50
NUM_SAMPLES = 20
B = 128
N = 8192
VS = N - SEG_LEN

_NC = 2
_NS = 16
_NW = _NC * _NS
_SEGS = NUM_SAMPLES * B
_SPW = _SEGS // _NW
_TROWS = B * N * 2 // 128
_SOFF = 128

_SUB = 64
_LANE = 128


_BB = 8


def _tc_body(xp_ref, g_ref, w1_ref, b1_ref, w2_ref, b2_ref, w3_ref, b3_ref,
             starts_ref, tbl_ref):
    rows = []
    for r in range(_BB):
        xp = xp_ref[r]

        tbl_ref[pl.ds(r * 128, 128)] = jnp.reshape(xp[:, 2:N + 2], (128, 128))

        j4 = lax.broadcasted_iota(jnp.int32, (2, N + 4), 1)
        xz = jnp.where((j4 >= 2) & (j4 < N + 2), xp, 0.0)
        x10 = jnp.concatenate([xz[:, k:k + N] for k in range(5)], axis=0)
        h1 = jnp.dot(w1_ref[...], x10, preferred_element_type=jnp.float32)
        h1 = jnp.maximum(h1 + b1_ref[...], 0.0)

        p = jnp.dot(w2_ref[...], h1, preferred_element_type=jnp.float32)
        ln = lax.broadcasted_iota(jnp.int32, (32, N), 1)
        acc = (p[32:64]
               + jnp.where(ln == 0, 0.0, pltpu.roll(p[0:32], 1, 1))
               + jnp.where(ln == N - 1, 0.0, pltpu.roll(p[64:96], N - 1, 1)))
        h2 = jnp.maximum(acc + b2_ref[...], 0.0)

        imp = jax.nn.sigmoid(jnp.dot(w3_ref[...], h2,
                                     preferred_element_type=jnp.float32)
                             + b3_ref[...])

        fwd = xp[:, 3:N + 3] - xp[:, 2:N + 2]
        bwd = xp[:, 2:N + 2] - xp[:, 1:N + 1]
        cross = fwd[0:1] * bwd[1:2] - fwd[1:2] * bwd[0:1]
        rows.append(imp + 0.3 * jnp.abs(cross))

    comb = jnp.concatenate(rows, axis=0)
    lin = lax.broadcasted_iota(jnp.int32, (_BB, N), 1)
    linv = lax.broadcasted_iota(jnp.int32, (_BB, VS), 1)
    l128 = lax.broadcasted_iota(jnp.int32, (_BB, _LANE), 1)
    sv = jnp.zeros((_BB, _LANE), jnp.int32)
    for i in range(NUM_SAMPLES):
        score = comb[:, :VS] + g_ref[i]
        m = jnp.max(score, axis=1, keepdims=True)
        start = jnp.min(jnp.where(score == m, linv, jnp.int32(2 ** 30)),
                        axis=1, keepdims=True)
        sv = sv + jnp.where(l128 == i, start, 0)
        seg_mask = (lin >= start) & (lin < start + SEG_LEN)
        comb = jnp.where(seg_mask, comb * 0.1, comb)
    starts_ref[...] = sv


def _tc_starts(xp, gp, w1r, b1c, w2r, b2c, w3c, b3c):
    grid = (B // _BB,)
    return pl.pallas_call(
        _tc_body,
        grid=grid,
        in_specs=[
            pl.BlockSpec((_BB, 2, N + 4), lambda b: (b, 0, 0)),
            pl.BlockSpec((NUM_SAMPLES, _BB, VS), lambda b: (0, b, 0)),
            pl.BlockSpec((64, 10), lambda b: (0, 0)),
            pl.BlockSpec((64, 1), lambda b: (0, 0)),
            pl.BlockSpec((96, 64), lambda b: (0, 0)),
            pl.BlockSpec((32, 1), lambda b: (0, 0)),
            pl.BlockSpec((1, 32), lambda b: (0, 0)),
            pl.BlockSpec((1, 1), lambda b: (0, 0)),
        ],
        out_specs=[
            pl.BlockSpec((_BB, _LANE), lambda b: (b, 0)),
            pl.BlockSpec((_BB * 128, 128), lambda b: (b, 0)),
        ],
        out_shape=[
            jax.ShapeDtypeStruct((B, _LANE), jnp.int32),
            jax.ShapeDtypeStruct((_TROWS, 128), jnp.float32),
        ],
    )(xp, gp, w1r, b1c, w2r, b2c, w3c, b3c)


def _sc_gather_body(pts_hbm, idx_hbm, ext_hbm, out_hbm,
                    idx_v, ext_v, stage, rows_v, sem):
    wid = lax.axis_index("s") * _NC + lax.axis_index("c")
    base = wid * _SPW
    pltpu.sync_copy(idx_hbm.at[wid], idx_v)
    pltpu.sync_copy(ext_hbm.at[pl.ds(base, _SPW)], ext_v)
    cps = [
        pltpu.async_copy(pts_hbm.at[idx_v.at[bank]],
                         stage.at[pl.ds(bank * _SOFF, 128)], sem)
        for bank in range(4)
    ]
    for cp in cps:
        cp.wait()

    def seg_body(s, _):
        for c in range(8):
            lin = ext_v[s, pl.ds(c * 16, 16)]
            er = lax.shift_right_logical(lin, 7)
            ec = lax.bitwise_and(lin, 127)
            rows_v[s, pl.ds(c * 16, 16)] = plsc.load_gather(stage, [er, ec])
        return 0

    lax.fori_loop(0, _SPW, seg_body, 0)
    pltpu.sync_copy(rows_v, out_hbm.at[pl.ds(base, _SPW)])


@functools.cache
def _sc_gather_kernel():
    return pl.kernel(
        _sc_gather_body,
        mesh=plsc.VectorSubcoreMesh(core_axis_name="c", subcore_axis_name="s"),
        out_type=jax.ShapeDtypeStruct((_SEGS, 128), jnp.float32),
        scratch_types=[
            pltpu.VMEM((4, 128), jnp.int32),
            pltpu.VMEM((_SPW, 128), jnp.int32),
            pltpu.VMEM((4 * _SOFF, 128), jnp.float32),
            pltpu.VMEM((_SPW, 128), jnp.float32),
            pltpu.SemaphoreType.DMA,
        ],
        compiler_params=pltpu.CompilerParams(needs_layout_passes=False),
    )


def kernel(points, w1, b1, w2, b2, w3, b3):
    skey = jax.random.key(42)
    keys = jax.vmap(lambda i: jax.random.fold_in(skey, i))(
        jnp.arange(NUM_SAMPLES))
    gp = jax.vmap(lambda k: jax.random.gumbel(k, (B, VS), jnp.float32))(keys)

    xt = jnp.transpose(points, (0, 2, 1))
    xp = jnp.concatenate([xt[:, :, -2:], xt, xt[:, :, :2]], axis=2)

    w1r = jnp.transpose(w1, (0, 2, 1)).reshape(64, 10)
    w2r = jnp.transpose(w2, (2, 0, 1)).reshape(96, 64)
    b1c = b1.reshape(64, 1)
    b2c = b2.reshape(32, 1)
    w3c = w3.reshape(1, 32)
    b3c = b3.reshape(1, 1)

    starts2, table = _tc_starts(xp, gp, w1r, b1c, w2r, b2c, w3c, b3c)
    starts = starts2[:, :NUM_SAMPLES].T

    sflat = starts.reshape(_SEGS)
    bno = jnp.tile(jnp.arange(B, dtype=jnp.int32), (NUM_SAMPLES,))
    blk = sflat // 128
    o = sflat % 128
    need2 = (o >= 128 - SEG_LEN + 1).astype(jnp.int32)
    xr0 = bno * 128 + blk
    banks = jnp.stack([xr0, xr0 + need2, xr0 + 64, xr0 + 64 + need2],
                      axis=0)
    idxw = banks.reshape(4, _NW, _SPW).transpose(1, 0, 2)
    idxh = jnp.pad(idxw, ((0, 0), (0, 0), (0, 128 - _SPW)), mode="edge")

    s_local = jnp.arange(_SEGS, dtype=jnp.int32) % _SPW
    j = jnp.minimum(jnp.arange(128, dtype=jnp.int32), 99)
    p = j // 2
    c = j & 1
    pos = o[:, None] + p[None, :]
    bank = 2 * c[None, :] + (pos >= 128).astype(jnp.int32)
    ext = 128 * (128 * bank + s_local[:, None]) + (pos & 127)

    out = _sc_gather_kernel()(table, idxh, ext)
    return out[:, :100].reshape(NUM_SAMPLES, B, SEG_LEN, 2)


# speedup vs baseline: 2.2569x; 1.0501x over previous
import functools

import jax
import jax.numpy as jnp
from jax import lax
from jax.experimental import pallas as pl
from jax.experimental.pallas import tpu as pltpu
from jax.experimental.pallas import tpu_sc as plsc

SEG_LEN = 50
NUM_SAMPLES = 20
B = 128
N = 8192
VS = N - SEG_LEN

_NC = 2
_NS = 16
_NW = _NC * _NS
_SEGS = NUM_SAMPLES * B
_SPW = _SEGS // _NW
_TROWS = B * N * 2 // 128
_SOFF = 128

_SUB = 64
_LANE = 128


_BB = 16


def _tc_body(xp_ref, g_ref, w1_ref, b1_ref, w2_ref, b2_ref, w3_ref, b3_ref,
             starts_ref, tbl_ref):
    rows = []
    for r in range(_BB):
        xp = xp_ref[r]

        tbl_ref[pl.ds(r * 128, 128)] = jnp.reshape(xp[:, 2:N + 2], (128, 128))

        j4 = lax.broadcasted_iota(jnp.int32, (2, N + 4), 1)
        xz = jnp.where((j4 >= 2) & (j4 < N + 2), xp, 0.0)
        x10 = jnp.concatenate([xz[:, k:k + N] for k in range(5)], axis=0)
        h1 = jnp.dot(w1_ref[...], x10, preferred_element_type=jnp.float32)
        h1 = jnp.maximum(h1 + b1_ref[...], 0.0)

        p = jnp.dot(w2_ref[...], h1, preferred_element_type=jnp.float32)
        ln = lax.broadcasted_iota(jnp.int32, (32, N), 1)
        acc = (p[32:64]
               + jnp.where(ln == 0, 0.0, pltpu.roll(p[0:32], 1, 1))
               + jnp.where(ln == N - 1, 0.0, pltpu.roll(p[64:96], N - 1, 1)))
        h2 = jnp.maximum(acc + b2_ref[...], 0.0)

        imp = jax.nn.sigmoid(jnp.dot(w3_ref[...], h2,
                                     preferred_element_type=jnp.float32)
                             + b3_ref[...])

        fwd = xp[:, 3:N + 3] - xp[:, 2:N + 2]
        bwd = xp[:, 2:N + 2] - xp[:, 1:N + 1]
        cross = fwd[0:1] * bwd[1:2] - fwd[1:2] * bwd[0:1]
        rows.append(imp + 0.3 * jnp.abs(cross))

    comb = jnp.concatenate(rows, axis=0)
    lin = lax.broadcasted_iota(jnp.int32, (_BB, N), 1)
    linv = lax.broadcasted_iota(jnp.int32, (_BB, VS), 1)
    l128 = lax.broadcasted_iota(jnp.int32, (_BB, _LANE), 1)
    sv = jnp.zeros((_BB, _LANE), jnp.int32)
    for i in range(NUM_SAMPLES):
        score = comb[:, :VS] + g_ref[i]
        m = jnp.max(score, axis=1, keepdims=True)
        start = jnp.min(jnp.where(score == m, linv, jnp.int32(2 ** 30)),
                        axis=1, keepdims=True)
        sv = sv + jnp.where(l128 == i, start, 0)
        seg_mask = (lin >= start) & (lin < start + SEG_LEN)
        comb = jnp.where(seg_mask, comb * 0.1, comb)
    starts_ref[...] = sv


def _tc_starts(xp, gp, w1r, b1c, w2r, b2c, w3c, b3c):
    grid = (B // _BB,)
    return pl.pallas_call(
        _tc_body,
        grid=grid,
        in_specs=[
            pl.BlockSpec((_BB, 2, N + 4), lambda b: (b, 0, 0)),
            pl.BlockSpec((NUM_SAMPLES, _BB, VS), lambda b: (0, b, 0)),
            pl.BlockSpec((64, 10), lambda b: (0, 0)),
            pl.BlockSpec((64, 1), lambda b: (0, 0)),
            pl.BlockSpec((96, 64), lambda b: (0, 0)),
            pl.BlockSpec((32, 1), lambda b: (0, 0)),
            pl.BlockSpec((1, 32), lambda b: (0, 0)),
            pl.BlockSpec((1, 1), lambda b: (0, 0)),
        ],
        out_specs=[
            pl.BlockSpec((_BB, _LANE), lambda b: (b, 0)),
            pl.BlockSpec((_BB * 128, 128), lambda b: (b, 0)),
        ],
        out_shape=[
            jax.ShapeDtypeStruct((B, _LANE), jnp.int32),
            jax.ShapeDtypeStruct((_TROWS, 128), jnp.float32),
        ],
    )(xp, gp, w1r, b1c, w2r, b2c, w3c, b3c)


def _sc_gather_body(pts_hbm, idx_hbm, ext_hbm, out_hbm,
                    idx_v, ext_v, stage, rows_v, sem):
    wid = lax.axis_index("s") * _NC + lax.axis_index("c")
    base = wid * _SPW
    pltpu.sync_copy(idx_hbm.at[wid], idx_v)
    pltpu.sync_copy(ext_hbm.at[pl.ds(base, _SPW)], ext_v)
    cps = [
        pltpu.async_copy(pts_hbm.at[idx_v.at[bank]],
                         stage.at[pl.ds(bank * _SOFF, 128)], sem)
        for bank in range(4)
    ]
    for cp in cps:
        cp.wait()

    def seg_body(s, _):
        for c in range(8):
            lin = ext_v[s, pl.ds(c * 16, 16)]
            er = lax.shift_right_logical(lin, 7)
            ec = lax.bitwise_and(lin, 127)
            rows_v[s, pl.ds(c * 16, 16)] = plsc.load_gather(stage, [er, ec])
        return 0

    lax.fori_loop(0, _SPW, seg_body, 0)
    pltpu.sync_copy(rows_v, out_hbm.at[pl.ds(base, _SPW)])


@functools.cache
def _sc_gather_kernel():
    return pl.kernel(
        _sc_gather_body,
        mesh=plsc.VectorSubcoreMesh(core_axis_name="c", subcore_axis_name="s"),
        out_type=jax.ShapeDtypeStruct((_SEGS, 128), jnp.float32),
        scratch_types=[
            pltpu.VMEM((4, 128), jnp.int32),
            pltpu.VMEM((_SPW, 128), jnp.int32),
            pltpu.VMEM((4 * _SOFF, 128), jnp.float32),
            pltpu.VMEM((_SPW, 128), jnp.float32),
            pltpu.SemaphoreType.DMA,
        ],
        compiler_params=pltpu.CompilerParams(needs_layout_passes=False),
    )


def kernel(points, w1, b1, w2, b2, w3, b3):
    skey = jax.random.key(42)
    keys = jax.vmap(lambda i: jax.random.fold_in(skey, i))(
        jnp.arange(NUM_SAMPLES))
    gp = jax.vmap(lambda k: jax.random.gumbel(k, (B, VS), jnp.float32))(keys)

    xt = jnp.transpose(points, (0, 2, 1))
    xp = jnp.concatenate([xt[:, :, -2:], xt, xt[:, :, :2]], axis=2)

    w1r = jnp.transpose(w1, (0, 2, 1)).reshape(64, 10)
    w2r = jnp.transpose(w2, (2, 0, 1)).reshape(96, 64)
    b1c = b1.reshape(64, 1)
    b2c = b2.reshape(32, 1)
    w3c = w3.reshape(1, 32)
    b3c = b3.reshape(1, 1)

    starts2, table = _tc_starts(xp, gp, w1r, b1c, w2r, b2c, w3c, b3c)
    starts = starts2[:, :NUM_SAMPLES].T

    sflat = starts.reshape(_SEGS)
    bno = jnp.tile(jnp.arange(B, dtype=jnp.int32), (NUM_SAMPLES,))
    blk = sflat // 128
    o = sflat % 128
    need2 = (o >= 128 - SEG_LEN + 1).astype(jnp.int32)
    xr0 = bno * 128 + blk
    banks = jnp.stack([xr0, xr0 + need2, xr0 + 64, xr0 + 64 + need2],
                      axis=0)
    idxw = banks.reshape(4, _NW, _SPW).transpose(1, 0, 2)
    idxh = jnp.pad(idxw, ((0, 0), (0, 0), (0, 128 - _SPW)), mode="edge")

    s_local = jnp.arange(_SEGS, dtype=jnp.int32) % _SPW
    j = jnp.minimum(jnp.arange(128, dtype=jnp.int32), 99)
    p = j // 2
    c = j & 1
    pos = o[:, None] + p[None, :]
    bank = 2 * c[None, :] + (pos >= 128).astype(jnp.int32)
    ext = 128 * (128 * bank + s_local[:, None]) + (pos & 127)

    out = _sc_gather_kernel()(table, idxh, ext)
    return out[:, :100].reshape(NUM_SAMPLES, B, SEG_LEN, 2)
